# baseline (speedup 1.0000x reference)
import functools

import jax
import jax.numpy as jnp
import numpy as np
from jax import lax
from jax.experimental import pallas as pl
from jax.experimental.pallas import tpu as pltpu
from jax.experimental.pallas import tpu_sc as plsc

B = 128
N = 32768
L = 16
NC = 2
NS = 16
NW = NC * NS
RPW = B // NW
NV = N // L
NBINS = 256
HIST_WORDS = NBINS
UNROLL = 16

_I32_MIN = np.int32(-(2 ** 31))


def _bkey(v):
    b = lax.bitcast_convert_type(v, jnp.int32)
    m = lax.shift_right_arithmetic(b, 31)
    return lax.bitwise_xor(b, lax.bitwise_or(m, _I32_MIN))


def _body(scores_hbm, params_hbm, out_hbm,
          buf0, buf1, keys, hist, params_v,
          isem0, isem1, osem0, osem1):
    wid = lax.axis_index("s") * NC + lax.axis_index("c")
    row0 = wid * RPW

    pltpu.sync_copy(params_hbm, params_v)
    k_lo = params_v[pl.ds(0, L)]
    k_hi = params_v[pl.ds(L, L)]

    zeros = jnp.zeros((L,), jnp.int32)
    idx15 = jnp.full((L,), L - 1, jnp.int32)
    r1_val = jnp.full((L,), 65537, jnp.int32)

    @plsc.parallel_loop(0, HIST_WORDS // L, unroll=4)
    def _(i):
        hist[pl.ds(i * L, L)] = zeros

    def merge_scan(kres_lo, kres_hi):
        def v_body(v, c):
            carry_lo, nd_lo, rm_lo, carry_hi, nd_hi, rm_hi = c
            acc = hist[pl.ds(v * L, L)]
            hist[pl.ds(v * L, L)] = zeros
            acc_lo = lax.bitwise_and(acc, jnp.int32(0xFFFF))
            acc_hi = lax.shift_right_logical(acc, 16)
            cum_lo = plsc.cumsum(acc_lo) + carry_lo
            cum_hi = plsc.cumsum(acc_hi) + carry_hi
            m_lo = cum_lo < kres_lo
            m_hi = cum_hi < kres_hi
            nd_lo = nd_lo + plsc.all_reduce_population_count(m_lo)
            nd_hi = nd_hi + plsc.all_reduce_population_count(m_hi)
            rm_lo = jnp.maximum(rm_lo, jnp.where(m_lo, cum_lo, 0))
            rm_hi = jnp.maximum(rm_hi, jnp.where(m_hi, cum_hi, 0))
            carry_lo = cum_lo[idx15]
            carry_hi = cum_hi[idx15]
            return carry_lo, nd_lo, rm_lo, carry_hi, nd_hi, rm_hi

        init = (zeros, zeros, zeros, zeros, zeros, zeros)
        _, nd_lo, rm_lo, _, nd_hi, rm_hi = plsc.parallel_loop(
            0, NBINS // L, unroll=4, carry=init)(v_body)
        cb_lo = jnp.broadcast_to(jnp.max(rm_lo), (L,))
        cb_hi = jnp.broadcast_to(jnp.max(rm_hi), (L,))
        return nd_lo, cb_lo, nd_hi, cb_hi

    def process(buf):
        @plsc.parallel_loop(0, NV, unroll=UNROLL)
        def _(j):
            off = j * L
            key = _bkey(buf[pl.ds(off, L)])
            keys[pl.ds(off, L)] = key
            addr = lax.shift_right_logical(key, 24)
            plsc.addupdate_scatter(hist, [addr], r1_val)

        d_lo, cb_lo, d_hi, cb_hi = merge_scan(k_lo, k_hi)
        pref_lo = d_lo
        pref_hi = d_hi
        kres_lo = k_lo - cb_lo
        kres_hi = k_hi - cb_hi

        for sh in (16, 8, 0):
            @plsc.parallel_loop(0, NV, unroll=UNROLL)
            def _(j, sh=sh, pl_v=pref_lo, ph_v=pref_hi):
                off = j * L
                t = lax.shift_right_logical(keys[pl.ds(off, L)], sh)
                pre = lax.shift_right_logical(t, 8)
                dig = lax.bitwise_and(t, jnp.int32(255))
                val = (jnp.where(pre == pl_v, jnp.int32(1), jnp.int32(0))
                       + jnp.where(pre == ph_v, jnp.int32(65536),
                                   jnp.int32(0)))
                plsc.addupdate_scatter(hist, [dig], val)
            d_lo, cb_lo, d_hi, cb_hi = merge_scan(kres_lo, kres_hi)
            pref_lo = lax.shift_left(pref_lo, 8) + d_lo
            pref_hi = lax.shift_left(pref_hi, 8) + d_hi
            kres_lo = kres_lo - cb_lo
            kres_hi = kres_hi - cb_hi

        thr_lo = lax.bitwise_xor(pref_lo, _I32_MIN)
        thr_hi = lax.bitwise_xor(pref_hi, _I32_MIN)

        @plsc.parallel_loop(0, NV, unroll=UNROLL)
        def _(j):
            off = j * L
            v = buf[pl.ds(off, L)]
            skey = lax.bitwise_xor(keys[pl.ds(off, L)], _I32_MIN)
            o = jnp.where(skey >= thr_hi, jnp.float32(1.0),
                          jnp.where(skey <= thr_lo, jnp.float32(0.0), v))
            buf[pl.ds(off, L)] = o

    bufs = (buf0, buf1)
    isems = (isem0, isem1)
    osems = (osem0, osem1)

    def in_copy(r):
        return pltpu.make_async_copy(
            scores_hbm.at[pl.ds((row0 + r) * N, N)], bufs[r % 2], isems[r % 2])

    def out_copy(r):
        return pltpu.make_async_copy(
            bufs[r % 2], out_hbm.at[pl.ds((row0 + r) * N, N)], osems[r % 2])

    in_copy(0).start()
    for r in range(RPW):
        in_copy(r).wait()
        if r + 1 < RPW:
            if r >= 1:
                out_copy(r - 1).wait()
            in_copy(r + 1).start()
        process(bufs[r % 2])
        out_copy(r).start()
    out_copy(RPW - 2).wait()
    out_copy(RPW - 1).wait()


@functools.partial(jax.jit, static_argnames=())
def _run(scores_flat, params):
    mesh = plsc.VectorSubcoreMesh(
        core_axis_name="c", subcore_axis_name="s",
        num_cores=NC, num_subcores=NS)
    fn = pl.kernel(
        _body,
        out_type=jax.ShapeDtypeStruct((B * N,), jnp.float32),
        mesh=mesh,
        compiler_params=pltpu.CompilerParams(needs_layout_passes=False),
        scratch_types=[
            pltpu.VMEM((N,), jnp.float32),
            pltpu.VMEM((N,), jnp.float32),
            pltpu.VMEM((N,), jnp.int32),
            pltpu.VMEM((HIST_WORDS,), jnp.int32),
            pltpu.VMEM((2 * L,), jnp.int32),
            pltpu.SemaphoreType.DMA,
            pltpu.SemaphoreType.DMA,
            pltpu.SemaphoreType.DMA,
            pltpu.SemaphoreType.DMA,
        ],
    )
    return fn(scores_flat, params)


def kernel(scores, k_remove, k_insert):
    k_lo = jnp.full((L,), k_remove, jnp.int32)
    k_hi = jnp.full((L,), N - jnp.asarray(k_insert, jnp.int32) + 1, jnp.int32)
    params = jnp.concatenate([k_lo, k_hi])
    out_flat = _run(jnp.reshape(scores, (B * N,)), params)
    return jnp.reshape(out_flat, (B, N))


# speedup vs baseline: 1.9944x; 1.9944x over previous
import functools

import jax
import jax.numpy as jnp
import numpy as np
from jax import lax
from jax.experimental import pallas as pl
from jax.experimental.pallas import tpu as pltpu
from jax.experimental.pallas import tpu_sc as plsc

B = 128
N = 32768
L = 16
NC = 2
NS = 16
NW = NC * NS
RPW = B // NW
NV = N // L
NBINS = 256
HIST_WORDS = NBINS
UNROLL = 8

_I32_MIN = np.int32(-(2 ** 31))


def _bkey(v):
    b = lax.bitcast_convert_type(v, jnp.int32)
    m = lax.shift_right_arithmetic(b, 31)
    return lax.bitwise_xor(b, lax.bitwise_or(m, _I32_MIN))


def _body(scores_hbm, params_hbm, out_hbm,
          buf0, buf1, keys, hist, params_v,
          isem0, isem1, osem0, osem1):
    wid = lax.axis_index("s") * NC + lax.axis_index("c")
    row0 = wid * RPW

    pltpu.sync_copy(params_hbm, params_v)
    k_lo = params_v[pl.ds(0, L)]
    k_hi = params_v[pl.ds(L, L)]

    zeros = jnp.zeros((L,), jnp.int32)
    idx15 = jnp.full((L,), L - 1, jnp.int32)
    r1_val = jnp.full((L,), 65537, jnp.int32)

    @plsc.parallel_loop(0, HIST_WORDS // L, unroll=4)
    def _(i):
        hist[pl.ds(i * L, L)] = zeros

    def merge_scan(kres_lo, kres_hi):
        def v_body(v, c):
            carry_lo, nd_lo, rm_lo, carry_hi, nd_hi, rm_hi = c
            acc = hist[pl.ds(v * L, L)]
            hist[pl.ds(v * L, L)] = zeros
            acc_lo = lax.bitwise_and(acc, jnp.int32(0xFFFF))
            acc_hi = lax.shift_right_logical(acc, 16)
            cum_lo = plsc.cumsum(acc_lo) + carry_lo
            cum_hi = plsc.cumsum(acc_hi) + carry_hi
            m_lo = cum_lo < kres_lo
            m_hi = cum_hi < kres_hi
            nd_lo = nd_lo + plsc.all_reduce_population_count(m_lo)
            nd_hi = nd_hi + plsc.all_reduce_population_count(m_hi)
            rm_lo = jnp.maximum(rm_lo, jnp.where(m_lo, cum_lo, 0))
            rm_hi = jnp.maximum(rm_hi, jnp.where(m_hi, cum_hi, 0))
            carry_lo = cum_lo[idx15]
            carry_hi = cum_hi[idx15]
            return carry_lo, nd_lo, rm_lo, carry_hi, nd_hi, rm_hi

        init = (zeros, zeros, zeros, zeros, zeros, zeros)
        _, nd_lo, rm_lo, _, nd_hi, rm_hi = plsc.parallel_loop(
            0, NBINS // L, unroll=4, carry=init)(v_body)
        cb_lo = jnp.broadcast_to(jnp.max(rm_lo), (L,))
        cb_hi = jnp.broadcast_to(jnp.max(rm_hi), (L,))
        return nd_lo, cb_lo, nd_hi, cb_hi

    def process(buf):
        @plsc.parallel_loop(0, NV, unroll=UNROLL)
        def _(j):
            off = j * L
            key = _bkey(buf[pl.ds(off, L)])
            keys[pl.ds(off, L)] = key
            addr = lax.shift_right_logical(key, 24)
            plsc.addupdate_scatter(hist, [addr], r1_val)

        d_lo, cb_lo, d_hi, cb_hi = merge_scan(k_lo, k_hi)
        pref_lo = d_lo
        pref_hi = d_hi
        kres_lo = k_lo - cb_lo
        kres_hi = k_hi - cb_hi

        for sh in (16, 8, 0):
            @plsc.parallel_loop(0, NV, unroll=UNROLL)
            def _(j, sh=sh, pl_v=pref_lo, ph_v=pref_hi):
                off = j * L
                t = lax.shift_right_logical(keys[pl.ds(off, L)], sh)
                pre = lax.shift_right_logical(t, 8)
                dig = lax.bitwise_and(t, jnp.int32(255))
                val = (jnp.where(pre == pl_v, jnp.int32(1), jnp.int32(0))
                       + jnp.where(pre == ph_v, jnp.int32(65536),
                                   jnp.int32(0)))
                plsc.addupdate_scatter(hist, [dig], val)
            d_lo, cb_lo, d_hi, cb_hi = merge_scan(kres_lo, kres_hi)
            pref_lo = lax.shift_left(pref_lo, 8) + d_lo
            pref_hi = lax.shift_left(pref_hi, 8) + d_hi
            kres_lo = kres_lo - cb_lo
            kres_hi = kres_hi - cb_hi

        thr_lo = lax.bitwise_xor(pref_lo, _I32_MIN)
        thr_hi = lax.bitwise_xor(pref_hi, _I32_MIN)

        @plsc.parallel_loop(0, NV, unroll=UNROLL)
        def _(j):
            off = j * L
            v = buf[pl.ds(off, L)]
            skey = lax.bitwise_xor(keys[pl.ds(off, L)], _I32_MIN)
            o = jnp.where(skey >= thr_hi, jnp.float32(1.0),
                          jnp.where(skey <= thr_lo, jnp.float32(0.0), v))
            buf[pl.ds(off, L)] = o

    bufs = (buf0, buf1)
    isems = (isem0, isem1)
    osems = (osem0, osem1)

    def in_copy(r):
        return pltpu.make_async_copy(
            scores_hbm.at[row0 + r], bufs[r % 2], isems[r % 2])

    def out_copy(r):
        return pltpu.make_async_copy(
            bufs[r % 2], out_hbm.at[row0 + r], osems[r % 2])

    in_copy(0).start()
    for r in range(RPW):
        in_copy(r).wait()
        if r + 1 < RPW:
            if r >= 1:
                out_copy(r - 1).wait()
            in_copy(r + 1).start()
        process(bufs[r % 2])
        out_copy(r).start()
    out_copy(RPW - 2).wait()
    out_copy(RPW - 1).wait()


@functools.partial(jax.jit, static_argnames=())
def _run(scores_flat, params):
    mesh = plsc.VectorSubcoreMesh(
        core_axis_name="c", subcore_axis_name="s",
        num_cores=NC, num_subcores=NS)
    fn = pl.kernel(
        _body,
        out_type=jax.ShapeDtypeStruct((B, N), jnp.float32),
        mesh=mesh,
        compiler_params=pltpu.CompilerParams(needs_layout_passes=False),
        scratch_types=[
            pltpu.VMEM((N,), jnp.float32),
            pltpu.VMEM((N,), jnp.float32),
            pltpu.VMEM((N,), jnp.int32),
            pltpu.VMEM((HIST_WORDS,), jnp.int32),
            pltpu.VMEM((2 * L,), jnp.int32),
            pltpu.SemaphoreType.DMA,
            pltpu.SemaphoreType.DMA,
            pltpu.SemaphoreType.DMA,
            pltpu.SemaphoreType.DMA,
        ],
    )
    return fn(scores_flat, params)


def kernel(scores, k_remove, k_insert):
    k_lo = jnp.full((L,), k_remove, jnp.int32)
    k_hi = jnp.full((L,), N - jnp.asarray(k_insert, jnp.int32) + 1, jnp.int32)
    params = jnp.concatenate([k_lo, k_hi])
    return _run(scores, params)


# speedup vs baseline: 2.7494x; 1.3785x over previous
import functools

import jax
import jax.numpy as jnp
import numpy as np
from jax import lax
from jax.experimental import pallas as pl
from jax.experimental.pallas import tpu as pltpu
from jax.experimental.pallas import tpu_sc as plsc

B = 128
N = 32768
L = 16
NC = 2
NS = 16
NW = NC * NS
RPW = B // NW
NV = N // L
NB12 = 4096
NB8 = 256
UNROLL = 8

_I32_MIN = np.int32(-(2 ** 31))


def _bkey(v):
    b = lax.bitcast_convert_type(v, jnp.int32)
    m = lax.shift_right_arithmetic(b, 31)
    return lax.bitwise_xor(b, lax.bitwise_or(m, _I32_MIN))


def _lo16(x):
    return lax.bitwise_and(x, jnp.int32(0xFFFF))


def _hi16(x):
    return lax.shift_right_logical(x, 16)


def _body(scores_hbm, params_hbm, out_hbm,
          buf0, buf1, keys, hist, cumbuf, tots, offs, params_v,
          isem0, isem1, osem0, osem1):
    wid = lax.axis_index("s") * NC + lax.axis_index("c")
    row0 = wid * RPW

    pltpu.sync_copy(params_hbm, params_v)
    k_lo = params_v[pl.ds(0, L)]
    k_hi = params_v[pl.ds(L, L)]

    zeros = jnp.zeros((L,), jnp.int32)
    lanes = lax.iota(jnp.int32, L)
    idx15 = jnp.full((L,), L - 1, jnp.int32)
    lane0 = lanes == 0
    r1_val = jnp.full((L,), 65537, jnp.int32)

    @plsc.parallel_loop(0, NB12 // L, unroll=4)
    def _(i):
        hist[pl.ds(i * L, L)] = zeros

    def big_scan(kres_lo, kres_hi):
        @plsc.parallel_loop(0, NB12 // L, unroll=4)
        def _(v):
            acc = hist[pl.ds(v * L, L)]
            hist[pl.ds(v * L, L)] = zeros
            cum = plsc.cumsum(acc)
            cumbuf[pl.ds(v * L, L)] = cum
            plsc.store_compressed(tots.at[pl.ds(v, L)], cum[idx15],
                                  mask=lane0)

        def b_body(w, c):
            carry, nv_lo, nv_hi = c
            tv = tots[pl.ds(w * L, L)]
            ctv = plsc.cumsum(tv) + carry
            offs[pl.ds(w * L, L)] = ctv - tv
            nv_lo = nv_lo + plsc.all_reduce_population_count(
                _lo16(ctv) < kres_lo)
            nv_hi = nv_hi + plsc.all_reduce_population_count(
                _hi16(ctv) < kres_hi)
            carry = ctv[idx15]
            return carry, nv_lo, nv_hi

        _, vs_lo, vs_hi = lax.fori_loop(
            0, NB12 // L // L, b_body, (zeros, zeros, zeros))

        def boundary(vstar, kres, unpack):
            cumv = unpack(plsc.load_gather(cumbuf, [vstar * L + lanes]))
            ev = unpack(plsc.load_gather(offs, [vstar]))
            c = cumv + ev
            m = c < kres
            d = vstar * L + plsc.all_reduce_population_count(m)
            cb_in = jnp.broadcast_to(jnp.max(jnp.where(m, c, 0)), (L,))
            return d, jnp.maximum(ev, cb_in)

        d_lo, cb_lo = boundary(vs_lo, kres_lo, _lo16)
        d_hi, cb_hi = boundary(vs_hi, kres_hi, _hi16)
        return d_lo, cb_lo, d_hi, cb_hi

    def small_scan(kres_lo, kres_hi):
        def v_body(v, c):
            carry_lo, nd_lo, rm_lo, carry_hi, nd_hi, rm_hi = c
            acc = hist[pl.ds(v * L, L)]
            hist[pl.ds(v * L, L)] = zeros
            cum_lo = plsc.cumsum(_lo16(acc)) + carry_lo
            cum_hi = plsc.cumsum(_hi16(acc)) + carry_hi
            m_lo = cum_lo < kres_lo
            m_hi = cum_hi < kres_hi
            nd_lo = nd_lo + plsc.all_reduce_population_count(m_lo)
            nd_hi = nd_hi + plsc.all_reduce_population_count(m_hi)
            rm_lo = jnp.maximum(rm_lo, jnp.where(m_lo, cum_lo, 0))
            rm_hi = jnp.maximum(rm_hi, jnp.where(m_hi, cum_hi, 0))
            carry_lo = cum_lo[idx15]
            carry_hi = cum_hi[idx15]
            return carry_lo, nd_lo, rm_lo, carry_hi, nd_hi, rm_hi

        init = (zeros, zeros, zeros, zeros, zeros, zeros)
        _, nd_lo, rm_lo, _, nd_hi, rm_hi = plsc.parallel_loop(
            0, NB8 // L, unroll=4, carry=init)(v_body)
        cb_lo = jnp.broadcast_to(jnp.max(rm_lo), (L,))
        cb_hi = jnp.broadcast_to(jnp.max(rm_hi), (L,))
        return nd_lo, cb_lo, nd_hi, cb_hi

    def process(buf):
        @plsc.parallel_loop(0, NV, unroll=UNROLL)
        def _(j):
            off = j * L
            key = _bkey(buf[pl.ds(off, L)])
            keys[pl.ds(off, L)] = key
            addr = lax.shift_right_logical(key, 20)
            plsc.addupdate_scatter(hist, [addr], r1_val)

        d_lo, cb_lo, d_hi, cb_hi = big_scan(k_lo, k_hi)
        pref_lo = d_lo
        pref_hi = d_hi
        kres_lo = k_lo - cb_lo
        kres_hi = k_hi - cb_hi

        @plsc.parallel_loop(0, NV, unroll=UNROLL)
        def _(j, pl_v=pref_lo, ph_v=pref_hi):
            off = j * L
            key = keys[pl.ds(off, L)]
            pre = lax.shift_right_logical(key, 20)
            dig = lax.bitwise_and(lax.shift_right_logical(key, 8),
                                  jnp.int32(0xFFF))
            val = (jnp.where(pre == pl_v, jnp.int32(1), jnp.int32(0))
                   + jnp.where(pre == ph_v, jnp.int32(65536), jnp.int32(0)))
            plsc.addupdate_scatter(hist, [dig], val)

        d_lo, cb_lo, d_hi, cb_hi = big_scan(kres_lo, kres_hi)
        pref_lo = lax.shift_left(pref_lo, 12) + d_lo
        pref_hi = lax.shift_left(pref_hi, 12) + d_hi
        kres_lo = kres_lo - cb_lo
        kres_hi = kres_hi - cb_hi

        @plsc.parallel_loop(0, NV, unroll=UNROLL)
        def _(j, pl_v=pref_lo, ph_v=pref_hi):
            off = j * L
            key = keys[pl.ds(off, L)]
            pre = lax.shift_right_logical(key, 8)
            dig = lax.bitwise_and(key, jnp.int32(0xFF))
            val = (jnp.where(pre == pl_v, jnp.int32(1), jnp.int32(0))
                   + jnp.where(pre == ph_v, jnp.int32(65536), jnp.int32(0)))
            plsc.addupdate_scatter(hist, [dig], val)

        d_lo, cb_lo, d_hi, cb_hi = small_scan(kres_lo, kres_hi)
        pref_lo = lax.shift_left(pref_lo, 8) + d_lo
        pref_hi = lax.shift_left(pref_hi, 8) + d_hi

        thr_lo = lax.bitwise_xor(pref_lo, _I32_MIN)
        thr_hi = lax.bitwise_xor(pref_hi, _I32_MIN)

        @plsc.parallel_loop(0, NV, unroll=UNROLL)
        def _(j):
            off = j * L
            v = buf[pl.ds(off, L)]
            skey = lax.bitwise_xor(keys[pl.ds(off, L)], _I32_MIN)
            o = jnp.where(skey >= thr_hi, jnp.float32(1.0),
                          jnp.where(skey <= thr_lo, jnp.float32(0.0), v))
            buf[pl.ds(off, L)] = o

    bufs = (buf0, buf1)
    isems = (isem0, isem1)
    osems = (osem0, osem1)

    def in_copy(r):
        return pltpu.make_async_copy(
            scores_hbm.at[row0 + r], bufs[r % 2], isems[r % 2])

    def out_copy(r):
        return pltpu.make_async_copy(
            bufs[r % 2], out_hbm.at[row0 + r], osems[r % 2])

    in_copy(0).start()
    for r in range(RPW):
        in_copy(r).wait()
        if r + 1 < RPW:
            if r >= 1:
                out_copy(r - 1).wait()
            in_copy(r + 1).start()
        process(bufs[r % 2])
        out_copy(r).start()
    out_copy(RPW - 2).wait()
    out_copy(RPW - 1).wait()


@functools.partial(jax.jit, static_argnames=())
def _run(scores, params):
    mesh = plsc.VectorSubcoreMesh(
        core_axis_name="c", subcore_axis_name="s",
        num_cores=NC, num_subcores=NS)
    fn = pl.kernel(
        _body,
        out_type=jax.ShapeDtypeStruct((B, N), jnp.float32),
        mesh=mesh,
        compiler_params=pltpu.CompilerParams(needs_layout_passes=False),
        scratch_types=[
            pltpu.VMEM((N,), jnp.float32),
            pltpu.VMEM((N,), jnp.float32),
            pltpu.VMEM((N,), jnp.int32),
            pltpu.VMEM((NB12,), jnp.int32),
            pltpu.VMEM((NB12,), jnp.int32),
            pltpu.VMEM((NB12 // L + L,), jnp.int32),
            pltpu.VMEM((NB12 // L + L,), jnp.int32),
            pltpu.VMEM((2 * L,), jnp.int32),
            pltpu.SemaphoreType.DMA,
            pltpu.SemaphoreType.DMA,
            pltpu.SemaphoreType.DMA,
            pltpu.SemaphoreType.DMA,
        ],
    )
    return fn(scores, params)


def kernel(scores, k_remove, k_insert):
    k_lo = jnp.full((L,), k_remove, jnp.int32)
    k_hi = jnp.full((L,), N - jnp.asarray(k_insert, jnp.int32) + 1, jnp.int32)
    params = jnp.concatenate([k_lo, k_hi])
    return _run(scores, params)
